# baseline (speedup 1.0000x reference)
import functools

import jax
import jax.numpy as jnp
from jax.experimental import pallas as pl
from jax.experimental.pallas import tpu as pltpu


def _sage_body(bm, adj_ref, x_ref, w_ref, b_ref, out_ref, y_ref):
    i = pl.program_id(0)
    f = x_ref.shape[1]
    dn = (((1,), (1,)), ((), ()))

    @pl.when(i == 0)
    def _():
        y_ref[...] = jax.lax.dot_general(
            x_ref[...], w_ref[:, f:], dn, preferred_element_type=jnp.float32
        )

    xi = x_ref[pl.ds(i * bm, bm), :]
    zi = (
        jax.lax.dot_general(
            xi, w_ref[:, :f], dn, preferred_element_type=jnp.float32
        )
        + b_ref[...]
    )
    out_ref[...] = zi + jnp.dot(
        adj_ref[...], y_ref[...], preferred_element_type=jnp.float32
    )


def _pick_block(n, target):
    for c in range(min(target, n), 7, -1):
        if n % c == 0 and c % 8 == 0:
            return c
    return n


def kernel(x, adj, W, b):
    n, f = x.shape
    bm = _pick_block(n, 80)
    out = pl.pallas_call(
        functools.partial(_sage_body, bm),
        grid=(n // bm,),
        in_specs=[
            pl.BlockSpec((bm, n), lambda i: (i, 0)),
            pl.BlockSpec((n, f), lambda i: (0, 0)),
            pl.BlockSpec(W.shape, lambda i: (0, 0)),
            pl.BlockSpec((1, f), lambda i: (0, 0)),
        ],
        out_specs=pl.BlockSpec((bm, f), lambda i: (i, 0)),
        out_shape=jax.ShapeDtypeStruct((n, f), jnp.float32),
        scratch_shapes=[pltpu.VMEM((n, f), jnp.float32)],
        compiler_params=pltpu.CompilerParams(
            dimension_semantics=("arbitrary",)
        ),
    )(adj, x, W, b.reshape(1, f))
    return out


# speedup vs baseline: 1.3528x; 1.3528x over previous
import functools

import jax
import jax.numpy as jnp
from jax.experimental import pallas as pl
from jax.experimental.pallas import tpu as pltpu


def _sage_body(bm, adj_ref, x_ref, w_ref, b_ref, out_ref, y_ref):
    i = pl.program_id(0)
    f = x_ref.shape[1]
    dn = (((1,), (1,)), ((), ()))

    @pl.when(i == 0)
    def _():
        y_ref[...] = jax.lax.dot_general(
            x_ref[...], w_ref[:, f:], dn, preferred_element_type=jnp.float32
        ).astype(jnp.bfloat16)

    xi = x_ref[pl.ds(i * bm, bm), :]
    zi = (
        jax.lax.dot_general(
            xi, w_ref[:, :f], dn, preferred_element_type=jnp.float32
        )
        + b_ref[...]
    )
    out_ref[...] = zi + jnp.dot(
        adj_ref[...].astype(jnp.bfloat16),
        y_ref[...],
        preferred_element_type=jnp.float32,
    )


def _pick_block(n, target):
    for c in range(min(target, n), 7, -1):
        if n % c == 0 and c % 8 == 0:
            return c
    return n


def kernel(x, adj, W, b):
    n, f = x.shape
    bm = _pick_block(n, 200)
    out = pl.pallas_call(
        functools.partial(_sage_body, bm),
        grid=(n // bm,),
        in_specs=[
            pl.BlockSpec((bm, n), lambda i: (i, 0)),
            pl.BlockSpec((n, f), lambda i: (0, 0)),
            pl.BlockSpec(W.shape, lambda i: (0, 0)),
            pl.BlockSpec((1, f), lambda i: (0, 0)),
        ],
        out_specs=pl.BlockSpec((bm, f), lambda i: (i, 0)),
        out_shape=jax.ShapeDtypeStruct((n, f), jnp.float32),
        scratch_shapes=[pltpu.VMEM((n, f), jnp.bfloat16)],
        compiler_params=pltpu.CompilerParams(
            dimension_semantics=("arbitrary",)
        ),
    )(adj, x, W, b.reshape(1, f))
    return out


# speedup vs baseline: 1.3700x; 1.0128x over previous
import functools

import jax
import jax.numpy as jnp
from jax.experimental import pallas as pl
from jax.experimental.pallas import tpu as pltpu


def _sage_body(bm, adj_ref, x_ref, w_ref, b_ref, out_ref, y_ref):
    i = pl.program_id(0)
    f = x_ref.shape[1]
    dn = (((1,), (1,)), ((), ()))

    @pl.when(i == 0)
    def _():
        y_ref[...] = jax.lax.dot_general(
            x_ref[...], w_ref[:, f:], dn, preferred_element_type=jnp.float32
        ).astype(jnp.bfloat16)

    xi = x_ref[pl.ds(i * bm, bm), :]
    zi = (
        jax.lax.dot_general(
            xi, w_ref[:, :f], dn, preferred_element_type=jnp.float32
        )
        + b_ref[...]
    )
    out_ref[...] = zi + jnp.dot(
        adj_ref[...].astype(jnp.bfloat16),
        y_ref[...],
        preferred_element_type=jnp.float32,
    )


def _pick_block(n, target):
    for c in range(min(target, n), 7, -1):
        if n % c == 0 and c % 8 == 0:
            return c
    return n


def kernel(x, adj, W, b):
    n, f = x.shape
    bm = _pick_block(n, 400)
    out = pl.pallas_call(
        functools.partial(_sage_body, bm),
        grid=(n // bm,),
        in_specs=[
            pl.BlockSpec((bm, n), lambda i: (i, 0)),
            pl.BlockSpec((n, f), lambda i: (0, 0)),
            pl.BlockSpec(W.shape, lambda i: (0, 0)),
            pl.BlockSpec((1, f), lambda i: (0, 0)),
        ],
        out_specs=pl.BlockSpec((bm, f), lambda i: (i, 0)),
        out_shape=jax.ShapeDtypeStruct((n, f), jnp.float32),
        scratch_shapes=[pltpu.VMEM((n, f), jnp.bfloat16)],
        compiler_params=pltpu.CompilerParams(
            dimension_semantics=("arbitrary",)
        ),
    )(adj, x, W, b.reshape(1, f))
    return out


# speedup vs baseline: 1.3720x; 1.0014x over previous
import functools

import jax
import jax.numpy as jnp
from jax.experimental import pallas as pl
from jax.experimental.pallas import tpu as pltpu


def _sage_body(bm, adj_ref, x_ref, w_ref, b_ref, out_ref, y_ref):
    i = pl.program_id(0)
    f = x_ref.shape[1]
    dn = (((1,), (1,)), ((), ()))

    @pl.when(i == 0)
    def _():
        y_ref[...] = jax.lax.dot_general(
            x_ref[...].astype(jnp.bfloat16),
            w_ref[:, f:].astype(jnp.bfloat16),
            dn,
            preferred_element_type=jnp.float32,
        ).astype(jnp.bfloat16)

    xi = x_ref[pl.ds(i * bm, bm), :]
    zi = (
        jax.lax.dot_general(
            xi, w_ref[:, :f], dn, preferred_element_type=jnp.float32
        )
        + b_ref[...]
    )
    out_ref[...] = zi + jnp.dot(
        adj_ref[...].astype(jnp.bfloat16),
        y_ref[...],
        preferred_element_type=jnp.float32,
    )


def _pick_block(n, target):
    for c in range(min(target, n), 7, -1):
        if n % c == 0 and c % 8 == 0:
            return c
    return n


def kernel(x, adj, W, b):
    n, f = x.shape
    bm = _pick_block(n, 400)
    out = pl.pallas_call(
        functools.partial(_sage_body, bm),
        grid=(n // bm,),
        in_specs=[
            pl.BlockSpec((bm, n), lambda i: (i, 0)),
            pl.BlockSpec((n, f), lambda i: (0, 0)),
            pl.BlockSpec(W.shape, lambda i: (0, 0)),
            pl.BlockSpec((1, f), lambda i: (0, 0)),
        ],
        out_specs=pl.BlockSpec((bm, f), lambda i: (i, 0)),
        out_shape=jax.ShapeDtypeStruct((n, f), jnp.float32),
        scratch_shapes=[pltpu.VMEM((n, f), jnp.bfloat16)],
        compiler_params=pltpu.CompilerParams(
            dimension_semantics=("arbitrary",)
        ),
    )(adj, x, W, b.reshape(1, f))
    return out
